# baseline (speedup 1.0000x reference)
import functools

import jax
import jax.numpy as jnp
from jax import lax
from jax.experimental import pallas as pl
from jax.experimental.pallas import tpu as pltpu
from jax.experimental.pallas import tpu_sc as plsc

_LAMBDA = 0.001
_L = 16
_NC = 2
_NS = 16
_NW = _NC * _NS
_B = 16384
_BPW = _B // _NW
_CHUNK = 128
_NCHUNK = _BPW // _CHUNK
_F = 16
_GROUPS = _BPW // _L


def _sc_body(u0_ref, i0_ref, r_ref, eu_ref, ei_ref, ub_ref, ib_ref, avg_ref,
             out_ref,
             uidx_v, iidx_v, r_v, urows_v, irows_v, ubias_v, ibias_v,
             avg_v, res_v, sem):
    c = lax.axis_index("c")
    s = lax.axis_index("s")
    wid = s * _NC + c

    pltpu.sync_copy(u0_ref.at[wid], uidx_v)
    pltpu.sync_copy(i0_ref.at[wid], iidx_v)
    pltpu.sync_copy(r_ref.at[wid], r_v)
    pltpu.sync_copy(avg_ref, avg_v)

    copies = []
    for j in range(_NCHUNK):
        sl = pl.ds(j * _CHUNK, _CHUNK)
        copies.append(pltpu.async_copy(eu_ref.at[uidx_v.at[j]], urows_v.at[sl], sem))
        copies.append(pltpu.async_copy(ei_ref.at[iidx_v.at[j]], irows_v.at[sl], sem))
        copies.append(pltpu.async_copy(ub_ref.at[uidx_v.at[j]], ubias_v.at[sl], sem))
        copies.append(pltpu.async_copy(ib_ref.at[iidx_v.at[j]], ibias_v.at[sl], sem))
    for cp in copies:
        cp.wait()

    lane = lax.iota(jnp.int32, _L)
    avgv = avg_v[...]
    zero = jnp.zeros((_L,), jnp.float32)

    def group_body(t, carry):
        sse, u2, i2 = carry
        base = t * _L
        cvec = (ubias_v[pl.ds(base, _L)] + ibias_v[pl.ds(base, _L)]
                + avgv - r_v[pl.ds(base, _L)])
        for k in range(_L):
            b = base + k
            u = urows_v[b, :]
            it = irows_v[b, :]
            v = u * it
            s = jnp.sum(v)
            e = s + cvec[k]
            sse = sse + e * e
            u2 = u2 + u * u
            i2 = i2 + it * it
        return sse, u2, i2

    sse_s, u2, i2 = lax.fori_loop(
        0, _GROUPS, group_body, (jnp.float32(0.0), zero, zero))

    u2_s = jnp.sum(u2)
    i2_s = jnp.sum(i2)
    res = jnp.where(lane == 0, sse_s,
                    jnp.where(lane == 1, u2_s,
                              jnp.where(lane == 2, i2_s, 0.0)))
    res_v[...] = res
    pltpu.sync_copy(res_v, out_ref.at[wid])


@jax.jit
def kernel(user0, item_i0, ratings, embed_user, embed_item, user_bias_w,
           item_bias_w, avg_rating):
    u0 = user0.reshape(_NW, _NCHUNK, _CHUNK)
    i0 = item_i0.reshape(_NW, _NCHUNK, _CHUNK)
    r = ratings.astype(jnp.float32).reshape(_NW, _BPW)
    ub = user_bias_w.reshape(-1)
    ib = item_bias_w.reshape(-1)
    avg16 = jnp.broadcast_to(avg_rating.astype(jnp.float32), (_L,))

    mesh = plsc.VectorSubcoreMesh(core_axis_name="c", subcore_axis_name="s")
    sc_call = pl.kernel(
        _sc_body,
        mesh=mesh,
        compiler_params=pltpu.CompilerParams(
            needs_layout_passes=False, use_tc_tiling_on_sc=False),
        out_type=jax.ShapeDtypeStruct((_NW, _L), jnp.float32),
        scratch_types=[
            pltpu.VMEM((_NCHUNK, _CHUNK), jnp.int32),
            pltpu.VMEM((_NCHUNK, _CHUNK), jnp.int32),
            pltpu.VMEM((_BPW,), jnp.float32),
            pltpu.VMEM((_BPW, _F), jnp.float32),
            pltpu.VMEM((_BPW, _F), jnp.float32),
            pltpu.VMEM((_BPW,), jnp.float32),
            pltpu.VMEM((_BPW,), jnp.float32),
            pltpu.VMEM((_L,), jnp.float32),
            pltpu.VMEM((_L,), jnp.float32),
            pltpu.SemaphoreType.DMA,
        ],
    )
    parts = sc_call(u0, i0, r, embed_user, embed_item, ub, ib, avg16)

    sse = parts[:, 0].sum()
    u2 = parts[:, 1].sum()
    i2 = parts[:, 2].sum()
    task_loss = sse / _B
    l2 = _LAMBDA * (u2 / (_B * _F)) + _LAMBDA * (i2 / (_B * _F))
    loss = task_loss + l2
    return (loss, task_loss, l2)
